# baseline (speedup 1.0000x reference)
import functools

import jax
import jax.numpy as jnp
from jax import lax
from jax.experimental import pallas as pl
from jax.experimental.pallas import tpu as pltpu
from jax.experimental.pallas import tpu_sc as plsc

_T, _H, _W = 16, 32, 32
_D3 = 256
_D = 3 * _D3
_NROW = _T + _H + _W
_NPOS = 4 * 8192
_NC, _NS, _L = 2, 16, 16
_NW = _NC * _NS
_PER_W = _NPOS // _NW
_CHUNK = 64
_NCH = _PER_W // _CHUNK
_SLOT = _CHUNK * _D
_NBUF = 2


def _emb_body(tab, ids, out, tabv, ids_v, dummyv, stage, ssem, wsem):
    cid = lax.axis_index("c")
    sid = lax.axis_index("s")
    wid = sid * _NC + cid
    base = wid * _PER_W

    pltpu.sync_copy(tab, tabv)
    pltpu.sync_copy(ids.at[pl.ds(base, _PER_W)], ids_v)

    def chunk_body(c, _):
        b = c & (_NBUF - 1)

        @pl.when(c >= _NBUF)
        def _wait_slot():
            pltpu.make_async_copy(
                stage.at[sid, b], out.at[pl.ds(0, _SLOT)], wsem).wait()

        for kk in range(_CHUNK // _L):
            pvec = ids_v[pl.ds(c * _CHUNK + kk * _L, _L)]
            r0v = pvec >> 10
            r1v = ((pvec >> 5) & (_H - 1)) + _T
            r2v = (pvec & (_W - 1)) + _T + _H
            for l in range(_L):
                o = pl.multiple_of((kk * _L + l) * _D, _D3)
                pltpu.make_async_copy(
                    tabv.at[r0v[l]],
                    stage.at[sid, b, pl.ds(o, _D3)], ssem).start()
                pltpu.make_async_copy(
                    tabv.at[r1v[l]],
                    stage.at[sid, b, pl.ds(o + _D3, _D3)], ssem).start()
                pltpu.make_async_copy(
                    tabv.at[r2v[l]],
                    stage.at[sid, b, pl.ds(o + 2 * _D3, _D3)], ssem).start()

        pltpu.make_async_copy(
            out.at[pl.ds(0, _SLOT)], dummyv, ssem).wait()
        pltpu.make_async_copy(
            stage.at[sid, b],
            out.at[pl.ds((base + c * _CHUNK) * _D, _SLOT)], wsem).start()
        return 0

    lax.fori_loop(0, _NCH, chunk_body, 0)

    for _ in range(_NBUF):
        pltpu.make_async_copy(
            stage.at[sid, 0], out.at[pl.ds(0, _SLOT)], wsem).wait()


@functools.partial(
    pl.kernel,
    mesh=plsc.VectorSubcoreMesh(core_axis_name="c", subcore_axis_name="s"),
    out_type=jax.ShapeDtypeStruct((_NPOS * _D,), jnp.float32),
    scratch_types=[
        pltpu.VMEM((_NROW, _D3), jnp.float32),
        pltpu.VMEM((_PER_W,), jnp.int32),
        pltpu.VMEM((_SLOT,), jnp.float32),
        pltpu.VMEM_SHARED((_NS, _NBUF, _SLOT), jnp.float32),
        pltpu.SemaphoreType.DMA,
        pltpu.SemaphoreType.DMA,
    ],
    compiler_params=pltpu.CompilerParams(needs_layout_passes=False),
)
def _emb_kernel(tab, ids, out, *scratch):
    _emb_body(tab, ids, out, *scratch)


def kernel(d_0, d_1, d_2, position_ids):
    B, Lseq = position_ids.shape
    ids = position_ids.reshape(-1).astype(jnp.int32)
    tab = jnp.concatenate([d_0, d_1, d_2], axis=0)
    out = _emb_kernel(tab, ids)
    return out.reshape(B, Lseq, _D)


# speedup vs baseline: 1.1374x; 1.1374x over previous
import functools

import jax
import jax.numpy as jnp
from jax import lax
from jax.experimental import pallas as pl
from jax.experimental.pallas import tpu as pltpu
from jax.experimental.pallas import tpu_sc as plsc

_T, _H, _W = 16, 32, 32
_D3 = 256
_D = 3 * _D3
_NROW = _T + _H + _W
_NPOS = 4 * 8192
_NC, _NS, _L = 2, 16, 16
_NW = _NC * _NS
_PER_W = _NPOS // _NW


def _emb_body(tab, ids, out, tabv, ids_v, dummyv, wsem):
    cid = lax.axis_index("c")
    sid = lax.axis_index("s")
    wid = sid * _NC + cid
    base = wid * _PER_W

    pltpu.sync_copy(tab, tabv)
    pltpu.sync_copy(ids.at[pl.ds(base, _PER_W)], ids_v)

    def group_body(g, _):
        @pl.when(g >= 1)
        def _drain_prev():
            pltpu.make_async_copy(
                out.at[pl.ds(0, _L * _D)], dummyv, wsem).wait()

        pvec = ids_v[pl.ds(g * _L, _L)]
        r0v = pvec >> 10
        r1v = ((pvec >> 5) & (_H - 1)) + _T
        r2v = (pvec & (_W - 1)) + _T + _H
        obase = (base + g * _L) * _D
        for l in range(_L):
            o = pl.multiple_of(obase + l * _D, _D3)
            pltpu.make_async_copy(
                tabv.at[r0v[l]], out.at[pl.ds(o, _D3)], wsem).start()
            pltpu.make_async_copy(
                tabv.at[r1v[l]], out.at[pl.ds(o + _D3, _D3)], wsem).start()
            pltpu.make_async_copy(
                tabv.at[r2v[l]], out.at[pl.ds(o + 2 * _D3, _D3)], wsem).start()
        return 0

    lax.fori_loop(0, _PER_W // _L, group_body, 0)

    pltpu.make_async_copy(
        out.at[pl.ds(0, _L * _D)], dummyv, wsem).wait()


@functools.partial(
    pl.kernel,
    mesh=plsc.VectorSubcoreMesh(core_axis_name="c", subcore_axis_name="s"),
    out_type=jax.ShapeDtypeStruct((_NPOS * _D,), jnp.float32),
    scratch_types=[
        pltpu.VMEM((_NROW, _D3), jnp.float32),
        pltpu.VMEM((_PER_W,), jnp.int32),
        pltpu.VMEM((_L * _D,), jnp.float32),
        pltpu.SemaphoreType.DMA,
    ],
    compiler_params=pltpu.CompilerParams(needs_layout_passes=False),
)
def _emb_kernel(tab, ids, out, *scratch):
    _emb_body(tab, ids, out, *scratch)


def kernel(d_0, d_1, d_2, position_ids):
    B, Lseq = position_ids.shape
    ids = position_ids.reshape(-1).astype(jnp.int32)
    tab = jnp.concatenate([d_0, d_1, d_2], axis=0)
    out = _emb_kernel(tab, ids)
    return out.reshape(B, Lseq, _D)


# speedup vs baseline: 3.1209x; 2.7439x over previous
import functools

import jax
import jax.numpy as jnp
from jax import lax
from jax.experimental import pallas as pl
from jax.experimental.pallas import tpu as pltpu
from jax.experimental.pallas import tpu_sc as plsc

_T, _H, _W = 16, 32, 32
_D3 = 256
_D = 3 * _D3
_NROW = _T + _H + _W
_NPOS = 4 * 8192
_NC, _NS, _L = 2, 16, 16
_NW = _NC * _NS
_PER_W = _NPOS // _NW


def _emb_body(tab, ids, out, tabv, ids_v, dummyv, wsem):
    cid = lax.axis_index("c")
    sid = lax.axis_index("s")
    wid = sid * _NC + cid
    base = wid * _PER_W

    pltpu.sync_copy(tab, tabv)
    pltpu.sync_copy(ids.at[pl.ds(base, _PER_W)], ids_v)

    def group_body(g, _):
        @pl.when(g >= 1)
        def _drain_prev():
            pltpu.make_async_copy(
                out.at[pl.ds(0, _L), :], dummyv, wsem).wait()

        pvec = ids_v[pl.ds(g * _L, _L)]
        r0v = pvec >> 10
        r1v = ((pvec >> 5) & (_H - 1)) + _T
        r2v = (pvec & (_W - 1)) + _T + _H
        rbase = base + g * _L
        for l in range(_L):
            row = rbase + l
            pltpu.make_async_copy(
                tabv.at[r0v[l]], out.at[row, pl.ds(0, _D3)], wsem).start()
            pltpu.make_async_copy(
                tabv.at[r1v[l]], out.at[row, pl.ds(_D3, _D3)], wsem).start()
            pltpu.make_async_copy(
                tabv.at[r2v[l]], out.at[row, pl.ds(2 * _D3, _D3)],
                wsem).start()
        return 0

    lax.fori_loop(0, _PER_W // _L, group_body, 0)

    pltpu.make_async_copy(
        out.at[pl.ds(0, _L), :], dummyv, wsem).wait()


@functools.partial(
    pl.kernel,
    mesh=plsc.VectorSubcoreMesh(core_axis_name="c", subcore_axis_name="s"),
    out_type=jax.ShapeDtypeStruct((_NPOS, _D), jnp.float32),
    scratch_types=[
        pltpu.VMEM((_NROW, _D3), jnp.float32),
        pltpu.VMEM((_PER_W,), jnp.int32),
        pltpu.VMEM((_L, _D), jnp.float32),
        pltpu.SemaphoreType.DMA,
    ],
    compiler_params=pltpu.CompilerParams(needs_layout_passes=False),
)
def _emb_kernel(tab, ids, out, *scratch):
    _emb_body(tab, ids, out, *scratch)


def kernel(d_0, d_1, d_2, position_ids):
    B, Lseq = position_ids.shape
    ids = position_ids.reshape(-1).astype(jnp.int32)
    tab = jnp.concatenate([d_0, d_1, d_2], axis=0)
    out = _emb_kernel(tab, ids)
    return out.reshape(B, Lseq, _D)


# speedup vs baseline: 3.3260x; 1.0657x over previous
import functools

import jax
import jax.numpy as jnp
from jax import lax
from jax.experimental import pallas as pl
from jax.experimental.pallas import tpu as pltpu
from jax.experimental.pallas import tpu_sc as plsc

_T, _H, _W = 16, 32, 32
_D3 = 256
_D = 3 * _D3
_NROW = _T + _H + _W
_NPOS = 4 * 8192
_NC, _NS, _L = 2, 16, 16
_NW = _NC * _NS
_PER_W = _NPOS // _NW


def _emb_body(tab, ids, out, tabv, ids_v, dummyv, wsem):
    cid = lax.axis_index("c")
    sid = lax.axis_index("s")
    wid = sid * _NC + cid
    base = wid * _PER_W

    pltpu.sync_copy(tab, tabv)
    pltpu.sync_copy(ids.at[pl.ds(base, _PER_W)], ids_v)

    def group_body(g, _):
        @pl.when(g >= 2)
        def _drain_prev():
            pltpu.make_async_copy(
                out.at[pl.ds(0, _L), :], dummyv, wsem).wait()

        pvec = ids_v[pl.ds(g * _L, _L)]
        r0v = pvec >> 10
        r1v = ((pvec >> 5) & (_H - 1)) + _T
        r2v = (pvec & (_W - 1)) + _T + _H
        rbase = base + g * _L
        for l in range(_L):
            row = rbase + l
            pltpu.make_async_copy(
                tabv.at[r0v[l]], out.at[row, pl.ds(0, _D3)], wsem).start()
            pltpu.make_async_copy(
                tabv.at[r1v[l]], out.at[row, pl.ds(_D3, _D3)], wsem).start()
            pltpu.make_async_copy(
                tabv.at[r2v[l]], out.at[row, pl.ds(2 * _D3, _D3)],
                wsem).start()
        return 0

    lax.fori_loop(0, _PER_W // _L, group_body, 0)

    pltpu.make_async_copy(
        out.at[pl.ds(0, _L), :], dummyv, wsem).wait()
    pltpu.make_async_copy(
        out.at[pl.ds(0, _L), :], dummyv, wsem).wait()


@functools.partial(
    pl.kernel,
    mesh=plsc.VectorSubcoreMesh(core_axis_name="c", subcore_axis_name="s"),
    out_type=jax.ShapeDtypeStruct((_NPOS, _D), jnp.float32),
    scratch_types=[
        pltpu.VMEM((_NROW, _D3), jnp.float32),
        pltpu.VMEM((_PER_W,), jnp.int32),
        pltpu.VMEM((_L, _D), jnp.float32),
        pltpu.SemaphoreType.DMA,
    ],
    compiler_params=pltpu.CompilerParams(needs_layout_passes=False),
)
def _emb_kernel(tab, ids, out, *scratch):
    _emb_body(tab, ids, out, *scratch)


def kernel(d_0, d_1, d_2, position_ids):
    B, Lseq = position_ids.shape
    ids = position_ids.reshape(-1).astype(jnp.int32)
    tab = jnp.concatenate([d_0, d_1, d_2], axis=0)
    out = _emb_kernel(tab, ids)
    return out.reshape(B, Lseq, _D)
